# Initial kernel scaffold; baseline (speedup 1.0000x reference)
import jax
import jax.numpy as jnp
from jax.experimental import pallas as pl


def kernel(input_ids, params):
    raise NotImplementedError("write your pallas kernel here")


# speedup vs baseline: 1.5296x; 1.5296x over previous
import functools

import jax
import jax.numpy as jnp
from jax import lax
from jax.experimental import pallas as pl
from jax.experimental.pallas import tpu as pltpu
from jax.experimental.pallas import tpu_sc as plsc

D = 768
DF = 4 * D
VOCAB = 21128
NR = 4
NS = 2
NE = NR + NS
S = 2048
NCHUNK = 1024
NN = DF // NCHUNK
VB = 2688
NV = (VOCAB + VB - 1) // VB
MB = 512
NM = S // MB


def _gelu(x):
    return x * 0.5 * (1.0 + lax.erf(x * 0.7071067811865476))


def _ln(x, g, b, eps):
    m = jnp.mean(x, axis=-1, keepdims=True)
    v = jnp.mean(jnp.square(x - m), axis=-1, keepdims=True)
    return (x - m) / jnp.sqrt(v + eps) * g + b


def _ln_na(x, eps):
    m = jnp.mean(x, axis=-1, keepdims=True)
    v = jnp.mean(jnp.square(x - m), axis=-1, keepdims=True)
    return (x - m) / jnp.sqrt(v + eps)


def _emb_gather(table, idx):
    info = plsc.get_sparse_core_info()
    nw = info.num_cores * info.num_subcores
    b_per_w = S // nw
    mesh = plsc.VectorSubcoreMesh(core_axis_name="c", subcore_axis_name="s")

    @functools.partial(
        pl.kernel, mesh=mesh,
        out_type=jax.ShapeDtypeStruct((S, D), jnp.float32),
        scratch_types=[
            pltpu.VMEM((b_per_w,), jnp.int32),
            pltpu.VMEM((b_per_w, D), jnp.float32),
            pltpu.SemaphoreType.DMA,
        ],
    )
    def k(table_hbm, idx_hbm, out_hbm, idx_v, rows_v, sem):
        wid = lax.axis_index("s") * info.num_cores + lax.axis_index("c")
        base = wid * b_per_w
        pltpu.sync_copy(idx_hbm.at[pl.ds(base, b_per_w)], idx_v)
        pltpu.async_copy(table_hbm.at[idx_v], rows_v, sem).wait()
        pltpu.sync_copy(rows_v, out_hbm.at[pl.ds(base, b_per_w)])

    return k(table, idx)


def _pre_body(rows_ref, pos_ref, g_ref, b_ref, o_ref):
    x = jnp.clip(rows_ref[...], -100.0, 100.0) + jnp.clip(pos_ref[...], -100.0, 100.0)
    o_ref[...] = _ln(x, g_ref[...], b_ref[...], 1e-5)


def _pre(rows, pos, g, b):
    return pl.pallas_call(
        _pre_body,
        out_shape=jax.ShapeDtypeStruct((S, D), jnp.float32),
    )(rows, pos, g.reshape(1, D), b.reshape(1, D))


def _router_body(h_ref, w1_ref, b1_ref, w2_ref, b2_ref, wall_ref):
    r = _gelu(jnp.dot(h_ref[...], w1_ref[...],
                      preferred_element_type=jnp.float32) + b1_ref[...])
    logits = jnp.dot(r, w2_ref[...], preferred_element_type=jnp.float32) + b2_ref[...]
    io = lax.broadcasted_iota(jnp.int32, (S, NR), 1)
    v1 = jnp.max(logits, axis=1, keepdims=True)
    i1 = jnp.min(jnp.where(logits == v1, io, NR), axis=1, keepdims=True)
    neg = jnp.where(io == i1, -1e30, logits)
    v2 = jnp.max(neg, axis=1, keepdims=True)
    i2 = jnp.min(jnp.where(neg == v2, io, NR), axis=1, keepdims=True)
    e2 = jnp.exp(v2 - v1)
    w1 = 1.0 / (1.0 + e2)
    w2 = e2 / (1.0 + e2)
    io6 = lax.broadcasted_iota(jnp.int32, (S, NE), 1)
    wall = (w1 * (io6 == i1) + w2 * (io6 == i2)
            + jnp.where(io6 >= NR, 1.0 / NS, 0.0))
    wall_ref[...] = wall


def _router(h, w1, b1, w2, b2):
    return pl.pallas_call(
        _router_body,
        out_shape=jax.ShapeDtypeStruct((S, NE), jnp.float32),
    )(h, w1, b1, w2, b2)


def _experts_body(h_ref, w1_ref, b1_ref, w2_ref, b2_ref, wall_ref,
                  out_ref, acc_ref):
    e = pl.program_id(0)
    n = pl.program_id(1)

    @pl.when(jnp.logical_and(e == 0, n == 0))
    def _():
        acc_ref[...] = jnp.zeros_like(acc_ref)

    t = _gelu(jnp.dot(h_ref[...], w1_ref[0],
                      preferred_element_type=jnp.float32) + b1_ref[0])
    part = jnp.dot(t, w2_ref[0], preferred_element_type=jnp.float32)
    io6 = lax.broadcasted_iota(jnp.int32, (S, NE), 1)
    w = jnp.sum(wall_ref[...] * (io6 == e), axis=1, keepdims=True)
    bias = jnp.where(n == 0, 1.0, 0.0)
    acc_ref[...] += w * (part + bias * b2_ref[0])

    @pl.when(jnp.logical_and(e == NE - 1, n == NN - 1))
    def _():
        out_ref[...] = acc_ref[...]


def _experts(h, w1s, b1s, w2s, b2s, wall):
    return pl.pallas_call(
        _experts_body,
        grid=(NE, NN),
        in_specs=[
            pl.BlockSpec((S, D), lambda e, n: (0, 0)),
            pl.BlockSpec((1, D, NCHUNK), lambda e, n: (e, 0, n)),
            pl.BlockSpec((1, 1, NCHUNK), lambda e, n: (e, 0, n)),
            pl.BlockSpec((1, NCHUNK, D), lambda e, n: (e, n, 0)),
            pl.BlockSpec((1, 1, D), lambda e, n: (e, 0, 0)),
            pl.BlockSpec((S, NE), lambda e, n: (0, 0)),
        ],
        out_specs=pl.BlockSpec((S, D), lambda e, n: (0, 0)),
        out_shape=jax.ShapeDtypeStruct((S, D), jnp.float32),
        scratch_shapes=[pltpu.VMEM((S, D), jnp.float32)],
    )(h, w1s, b1s, w2s, b2s, wall)


def _combine_body(h_ref, f_ref, w_ref, b_ref, g1_ref, bb1_ref, o_ref):
    out = jnp.dot(f_ref[...], w_ref[...], preferred_element_type=jnp.float32) + b_ref[...]
    out = out * 0.5 + h_ref[...] * 0.5
    a = _ln_na(out, 1e-6)
    o_ref[...] = _ln(h_ref[...] + a, g1_ref[...], bb1_ref[...], 1e-5)


def _combine(h, final, w, b, g1, b1):
    return pl.pallas_call(
        _combine_body,
        out_shape=jax.ShapeDtypeStruct((S, D), jnp.float32),
    )(h, final, w, b.reshape(1, D), g1.reshape(1, D), b1.reshape(1, D))


def _mlp_body(h_ref, wi_ref, bi_ref, wo_ref, bo_ref, g_ref, b_ref,
              o_ref, acc_ref):
    n = pl.program_id(0)

    @pl.when(n == 0)
    def _():
        acc_ref[...] = jnp.zeros_like(acc_ref)

    t = _gelu(jnp.dot(h_ref[...], wi_ref[...],
                      preferred_element_type=jnp.float32) + bi_ref[...])
    acc_ref[...] += jnp.dot(t, wo_ref[...], preferred_element_type=jnp.float32)

    @pl.when(n == NN - 1)
    def _():
        o = acc_ref[...] + bo_ref[...]
        o_ref[...] = _ln(h_ref[...] + o, g_ref[...], b_ref[...], 1e-5)


def _mlp(h, wi, bi, wo, bo, g, b):
    return pl.pallas_call(
        _mlp_body,
        grid=(NN,),
        in_specs=[
            pl.BlockSpec((S, D), lambda n: (0, 0)),
            pl.BlockSpec((D, NCHUNK), lambda n: (0, n)),
            pl.BlockSpec((1, NCHUNK), lambda n: (0, n)),
            pl.BlockSpec((NCHUNK, D), lambda n: (n, 0)),
            pl.BlockSpec((1, D), lambda n: (0, 0)),
            pl.BlockSpec((1, D), lambda n: (0, 0)),
            pl.BlockSpec((1, D), lambda n: (0, 0)),
        ],
        out_specs=pl.BlockSpec((S, D), lambda n: (0, 0)),
        out_shape=jax.ShapeDtypeStruct((S, D), jnp.float32),
        scratch_shapes=[pltpu.VMEM((S, D), jnp.float32)],
    )(h, wi, bi.reshape(1, DF), wo, bo.reshape(1, D), g.reshape(1, D),
      b.reshape(1, D))


def _lm_body(h_ref, w_ref, b_ref, o_ref):
    o_ref[...] = (jnp.dot(h_ref[...], w_ref[...],
                          preferred_element_type=jnp.float32) + b_ref[...])


def _lm_head(h, w, b):
    return pl.pallas_call(
        _lm_body,
        grid=(NV, NM),
        in_specs=[
            pl.BlockSpec((MB, D), lambda v, m: (m, 0)),
            pl.BlockSpec((D, VB), lambda v, m: (0, v)),
            pl.BlockSpec((1, VB), lambda v, m: (0, v)),
        ],
        out_specs=pl.BlockSpec((MB, VB), lambda v, m: (m, v)),
        out_shape=jax.ShapeDtypeStruct((S, VOCAB), jnp.float32),
    )(h, w, b.reshape(1, VOCAB))


def kernel(input_ids, params):
    p = params
    ids = input_ids.reshape(-1).astype(jnp.int32)
    rows = _emb_gather(p["emb"], ids)
    h = _pre(rows, p["pos"][:S], p["ln_g"], p["ln_b"])
    for lp in p["layers"]:
        wall = _router(h, lp["router1"]["w"], lp["router1"]["b"].reshape(1, D),
                       lp["router2"]["w"], lp["router2"]["b"].reshape(1, NR))
        experts = list(lp["routed"]) + list(lp["shared"])
        w1s = jnp.stack([e["l1"]["w"] for e in experts])
        b1s = jnp.stack([e["l1"]["b"].reshape(1, DF) for e in experts])
        w2s = jnp.stack([e["l2"]["w"] for e in experts])
        b2s = jnp.stack([e["l2"]["b"].reshape(1, D) for e in experts])
        final = _experts(h, w1s, b1s, w2s, b2s, wall)
        h = _combine(h, final, lp["attn_out"]["w"], lp["attn_out"]["b"],
                     lp["ln1_g"], lp["ln1_b"])
        h = _mlp(h, lp["inter"]["w"], lp["inter"]["b"], lp["out"]["w"],
                 lp["out"]["b"], lp["ln2_g"], lp["ln2_b"])
    logits = _lm_head(h, p["lm_head"]["w"], p["lm_head"]["b"])
    return logits.reshape(1, S, VOCAB)


# speedup vs baseline: 1.6466x; 1.0765x over previous
import functools

import jax
import jax.numpy as jnp
from jax import lax
from jax.experimental import pallas as pl
from jax.experimental.pallas import tpu as pltpu
from jax.experimental.pallas import tpu_sc as plsc

D = 768
DF = 4 * D
VOCAB = 21128
NR = 4
NS = 2
NE = NR + NS
S = 2048
NCHUNK = 1024
NN = DF // NCHUNK
VB = 2688
NV = (VOCAB + VB - 1) // VB
MB = 512
NM = S // MB


def _gelu(x):
    return x * 0.5 * (1.0 + lax.erf(x * 0.7071067811865476))


def _ln(x, g, b, eps):
    m = jnp.mean(x, axis=-1, keepdims=True)
    v = jnp.mean(jnp.square(x - m), axis=-1, keepdims=True)
    return (x - m) / jnp.sqrt(v + eps) * g + b


def _ln_na(x, eps):
    m = jnp.mean(x, axis=-1, keepdims=True)
    v = jnp.mean(jnp.square(x - m), axis=-1, keepdims=True)
    return (x - m) / jnp.sqrt(v + eps)


def _emb_gather(table, idx):
    info = plsc.get_sparse_core_info()
    nw = info.num_cores * info.num_subcores
    b_per_w = S // nw
    mesh = plsc.VectorSubcoreMesh(core_axis_name="c", subcore_axis_name="s")

    @functools.partial(
        pl.kernel, mesh=mesh,
        out_type=jax.ShapeDtypeStruct((S, D), jnp.float32),
        scratch_types=[
            pltpu.VMEM((b_per_w,), jnp.int32),
            pltpu.VMEM((b_per_w, D), jnp.float32),
            pltpu.SemaphoreType.DMA,
        ],
    )
    def k(table_hbm, idx_hbm, out_hbm, idx_v, rows_v, sem):
        wid = lax.axis_index("s") * info.num_cores + lax.axis_index("c")
        base = wid * b_per_w
        pltpu.sync_copy(idx_hbm.at[pl.ds(base, b_per_w)], idx_v)
        pltpu.async_copy(table_hbm.at[idx_v], rows_v, sem).wait()
        pltpu.sync_copy(rows_v, out_hbm.at[pl.ds(base, b_per_w)])

    return k(table, idx)


def _pre_body(rows_ref, pos_ref, g_ref, b_ref, o_ref):
    x = jnp.clip(rows_ref[...], -100.0, 100.0) + jnp.clip(pos_ref[...], -100.0, 100.0)
    o_ref[...] = _ln(x, g_ref[...], b_ref[...], 1e-5)


def _pre(rows, pos, g, b):
    return pl.pallas_call(
        _pre_body,
        out_shape=jax.ShapeDtypeStruct((S, D), jnp.float32),
    )(rows, pos, g.reshape(1, D), b.reshape(1, D))


def _router_body(h_ref, w1_ref, b1_ref, w2_ref, b2_ref, wall_ref):
    r = _gelu(jnp.dot(h_ref[...], w1_ref[...],
                      preferred_element_type=jnp.float32) + b1_ref[...])
    logits = jnp.dot(r, w2_ref[...], preferred_element_type=jnp.float32) + b2_ref[...]
    io = lax.broadcasted_iota(jnp.int32, (S, NR), 1)
    v1 = jnp.max(logits, axis=1, keepdims=True)
    i1 = jnp.min(jnp.where(logits == v1, io, NR), axis=1, keepdims=True)
    neg = jnp.where(io == i1, -1e30, logits)
    v2 = jnp.max(neg, axis=1, keepdims=True)
    i2 = jnp.min(jnp.where(neg == v2, io, NR), axis=1, keepdims=True)
    e2 = jnp.exp(v2 - v1)
    w1 = 1.0 / (1.0 + e2)
    w2 = e2 / (1.0 + e2)
    io6 = lax.broadcasted_iota(jnp.int32, (S, NE), 1)
    wall = (w1 * (io6 == i1) + w2 * (io6 == i2)
            + jnp.where(io6 >= NR, 1.0 / NS, 0.0))
    wall_ref[...] = wall


def _router(h, w1, b1, w2, b2):
    return pl.pallas_call(
        _router_body,
        out_shape=jax.ShapeDtypeStruct((S, NE), jnp.float32),
    )(h, w1, b1, w2, b2)


def _expert_first_body(h_ref, wall_ref, w1_ref, b1_ref, w2_ref, b2_ref,
                       out_ref, *, j):
    n = pl.program_id(0)
    t = _gelu(jnp.dot(h_ref[...], w1_ref[...],
                      preferred_element_type=jnp.float32) + b1_ref[...])
    part = jnp.dot(t, w2_ref[...], preferred_element_type=jnp.float32)
    io6 = lax.broadcasted_iota(jnp.int32, (S, NE), 1)
    w = jnp.sum(wall_ref[...] * (io6 == j), axis=1, keepdims=True)

    @pl.when(n == 0)
    def _():
        out_ref[...] = w * (part + b2_ref[...])

    @pl.when(n != 0)
    def _():
        out_ref[...] += w * part


def _expert_acc_body(h_ref, wall_ref, w1_ref, b1_ref, w2_ref, b2_ref,
                     acc_ref, out_ref, *, j):
    n = pl.program_id(0)
    t = _gelu(jnp.dot(h_ref[...], w1_ref[...],
                      preferred_element_type=jnp.float32) + b1_ref[...])
    part = jnp.dot(t, w2_ref[...], preferred_element_type=jnp.float32)
    io6 = lax.broadcasted_iota(jnp.int32, (S, NE), 1)
    w = jnp.sum(wall_ref[...] * (io6 == j), axis=1, keepdims=True)

    @pl.when(n == 0)
    def _():
        out_ref[...] = acc_ref[...] + w * (part + b2_ref[...])

    @pl.when(n != 0)
    def _():
        out_ref[...] += w * part


_EXPERT_SPECS = [
    pl.BlockSpec((S, D), lambda n: (0, 0)),
    pl.BlockSpec((S, NE), lambda n: (0, 0)),
    pl.BlockSpec((D, NCHUNK), lambda n: (0, n)),
    pl.BlockSpec((1, NCHUNK), lambda n: (0, n)),
    pl.BlockSpec((NCHUNK, D), lambda n: (n, 0)),
    pl.BlockSpec((1, D), lambda n: (0, 0)),
]


def _expert(h, wall, ex, j, acc):
    w1 = ex["l1"]["w"]
    b1 = ex["l1"]["b"].reshape(1, DF)
    w2 = ex["l2"]["w"]
    b2 = ex["l2"]["b"].reshape(1, D)
    if acc is None:
        return pl.pallas_call(
            functools.partial(_expert_first_body, j=j),
            grid=(NN,),
            in_specs=_EXPERT_SPECS,
            out_specs=pl.BlockSpec((S, D), lambda n: (0, 0)),
            out_shape=jax.ShapeDtypeStruct((S, D), jnp.float32),
        )(h, wall, w1, b1, w2, b2)
    return pl.pallas_call(
        functools.partial(_expert_acc_body, j=j),
        grid=(NN,),
        in_specs=_EXPERT_SPECS + [pl.BlockSpec((S, D), lambda n: (0, 0))],
        out_specs=pl.BlockSpec((S, D), lambda n: (0, 0)),
        out_shape=jax.ShapeDtypeStruct((S, D), jnp.float32),
        input_output_aliases={6: 0},
    )(h, wall, w1, b1, w2, b2, acc)


def _combine_body(h_ref, f_ref, w_ref, b_ref, g1_ref, bb1_ref, o_ref):
    out = jnp.dot(f_ref[...], w_ref[...], preferred_element_type=jnp.float32) + b_ref[...]
    out = out * 0.5 + h_ref[...] * 0.5
    a = _ln_na(out, 1e-6)
    o_ref[...] = _ln(h_ref[...] + a, g1_ref[...], bb1_ref[...], 1e-5)


def _combine(h, final, w, b, g1, b1):
    return pl.pallas_call(
        _combine_body,
        out_shape=jax.ShapeDtypeStruct((S, D), jnp.float32),
    )(h, final, w, b.reshape(1, D), g1.reshape(1, D), b1.reshape(1, D))


def _mlp_body(h_ref, wi_ref, bi_ref, wo_ref, bo_ref, g_ref, b_ref,
              o_ref, acc_ref):
    n = pl.program_id(0)

    @pl.when(n == 0)
    def _():
        acc_ref[...] = jnp.zeros_like(acc_ref)

    t = _gelu(jnp.dot(h_ref[...], wi_ref[...],
                      preferred_element_type=jnp.float32) + bi_ref[...])
    acc_ref[...] += jnp.dot(t, wo_ref[...], preferred_element_type=jnp.float32)

    @pl.when(n == NN - 1)
    def _():
        o = acc_ref[...] + bo_ref[...]
        o_ref[...] = _ln(h_ref[...] + o, g_ref[...], b_ref[...], 1e-5)


def _mlp(h, wi, bi, wo, bo, g, b):
    return pl.pallas_call(
        _mlp_body,
        grid=(NN,),
        in_specs=[
            pl.BlockSpec((S, D), lambda n: (0, 0)),
            pl.BlockSpec((D, NCHUNK), lambda n: (0, n)),
            pl.BlockSpec((1, NCHUNK), lambda n: (0, n)),
            pl.BlockSpec((NCHUNK, D), lambda n: (n, 0)),
            pl.BlockSpec((1, D), lambda n: (0, 0)),
            pl.BlockSpec((1, D), lambda n: (0, 0)),
            pl.BlockSpec((1, D), lambda n: (0, 0)),
        ],
        out_specs=pl.BlockSpec((S, D), lambda n: (0, 0)),
        out_shape=jax.ShapeDtypeStruct((S, D), jnp.float32),
        scratch_shapes=[pltpu.VMEM((S, D), jnp.float32)],
    )(h, wi, bi.reshape(1, DF), wo, bo.reshape(1, D), g.reshape(1, D),
      b.reshape(1, D))


def _lm_body(h_ref, w_ref, b_ref, o_ref):
    o_ref[...] = (jnp.dot(h_ref[...], w_ref[...],
                          preferred_element_type=jnp.float32) + b_ref[...])


def _lm_head(h, w, b):
    return pl.pallas_call(
        _lm_body,
        grid=(NV, NM),
        in_specs=[
            pl.BlockSpec((MB, D), lambda v, m: (m, 0)),
            pl.BlockSpec((D, VB), lambda v, m: (0, v)),
            pl.BlockSpec((1, VB), lambda v, m: (0, v)),
        ],
        out_specs=pl.BlockSpec((MB, VB), lambda v, m: (m, v)),
        out_shape=jax.ShapeDtypeStruct((S, VOCAB), jnp.float32),
    )(h, w, b.reshape(1, VOCAB))


def kernel(input_ids, params):
    p = params
    ids = input_ids.reshape(-1).astype(jnp.int32)
    rows = _emb_gather(p["emb"], ids)
    h = _pre(rows, p["pos"][:S], p["ln_g"], p["ln_b"])
    for lp in p["layers"]:
        wall = _router(h, lp["router1"]["w"], lp["router1"]["b"].reshape(1, D),
                       lp["router2"]["w"], lp["router2"]["b"].reshape(1, NR))
        experts = list(lp["routed"]) + list(lp["shared"])
        final = None
        for j, ex in enumerate(experts):
            final = _expert(h, wall, ex, j, final)
        h = _combine(h, final, lp["attn_out"]["w"], lp["attn_out"]["b"],
                     lp["ln1_g"], lp["ln1_b"])
        h = _mlp(h, lp["inter"]["w"], lp["inter"]["b"], lp["out"]["w"],
                 lp["out"]["b"], lp["ln2_g"], lp["ln2_b"])
    logits = _lm_head(h, p["lm_head"]["w"], p["lm_head"]["b"])
    return logits.reshape(1, S, VOCAB)
